# baseline (speedup 1.0000x reference)
import functools

import jax
import jax.numpy as jnp
from jax import lax
from jax.experimental import pallas as pl
from jax.experimental.pallas import tpu as pltpu
from jax.experimental.pallas import tpu_sc as plsc

N = 10000
E = 160000
D = 256
H = 512
G = 64
SF = 4

NC = 2
NS = 16
NW = NC * NS
WH = 128
TB = 128
NB = 40
E2 = NW * NB * TB
NP = 10240
STRIPE = NP // NS
ZB = 128
ZCH = STRIPE // ZB
CH = 8
NQ = NB // CH


def _sc_agg_body(P, with_cnt, *refs):
    if with_cnt:
        (table, src_off, dst_idx, zrows, zcnt, ones1,
         out_sums, out_cnt,
         srcv, dstv, rows, onesv, sems, csem, acc, cntacc) = refs
    else:
        (table, src_off, dst_idx, zrows,
         out_sums,
         srcv, dstv, rows, sems, acc) = refs

    c = lax.axis_index("c")
    s = lax.axis_index("s")
    tid = s * NC + c
    lo = s * STRIPE

    if with_cnt:
        pltpu.sync_copy(ones1, onesv)
        pltpu.sync_copy(zcnt.at[pl.ds(lo, STRIPE)],
                        cntacc.at[pl.ds(lo, STRIPE)])

    for p in range(P):
        for z in range(ZCH):
            pltpu.sync_copy(zrows, acc.at[pl.ds(lo + z * ZB, ZB)])
        plsc.subcore_barrier()

        do_cnt = with_cnt and p == 0
        gps = [None, None]
        sps = [None, None]
        cp = None
        pltpu.sync_copy(dst_idx.at[tid, pl.ds(0, CH)], dstv.at[0])
        pltpu.sync_copy(src_off.at[p, tid, pl.ds(0, CH)], srcv.at[0])
        gps[0] = pltpu.async_copy(table.at[srcv.at[0, 0]], rows.at[0],
                                  sems[0])
        for i in range(NB):
            q, b = divmod(i, CH)
            nq, nb = divmod(i + 1, CH)
            if nb == 0 and nq < NQ:
                for j in range(2):
                    if sps[j] is not None:
                        sps[j].wait()
                        sps[j] = None
                pltpu.sync_copy(dst_idx.at[tid, pl.ds(nq * CH, CH)],
                                dstv.at[nq % 2])
                pltpu.sync_copy(src_off.at[p, tid, pl.ds(nq * CH, CH)],
                                srcv.at[nq % 2])
            if i + 1 < NB:
                if sps[(i + 1) % 2] is not None:
                    sps[(i + 1) % 2].wait()
                    sps[(i + 1) % 2] = None
                gps[(i + 1) % 2] = pltpu.async_copy(
                    table.at[srcv.at[nq % 2, nb]], rows.at[(i + 1) % 2],
                    sems[(i + 1) % 2])
            gps[i % 2].wait()
            sps[i % 2] = pltpu.async_copy(rows.at[i % 2],
                                          acc.at[dstv.at[q % 2, b]],
                                          sems[2 + i % 2], add=True)
            if do_cnt:
                pltpu.sync_copy(onesv, cntacc.at[dstv.at[q % 2, b]],
                                add=True)
        for j in range(2):
            if sps[j] is not None:
                sps[j].wait()
        plsc.subcore_barrier()
        for z in range(ZCH):
            pltpu.sync_copy(acc.at[pl.ds(lo + z * ZB, ZB)],
                            out_sums.at[c, p, pl.ds(lo + z * ZB, ZB)])
        if do_cnt:
            pltpu.sync_copy(cntacc.at[pl.ds(lo, STRIPE)],
                            out_cnt.at[c, pl.ds(lo, STRIPE)])


def _make_sc_agg(P, with_cnt):
    out_type = [jax.ShapeDtypeStruct((NC, P, NP, WH), jnp.float32)]
    scratch = [
        pltpu.VMEM((2, CH, TB), jnp.int32),
        pltpu.VMEM((2, CH, TB), jnp.int32),
        pltpu.VMEM((2, TB, WH), jnp.float32),
    ]
    if with_cnt:
        out_type.append(jax.ShapeDtypeStruct((NC, NP), jnp.float32))
        scratch.append(pltpu.VMEM((TB,), jnp.float32))
    scratch.append([pltpu.SemaphoreType.DMA] * 4)
    if with_cnt:
        scratch.append(pltpu.SemaphoreType.DMA)
    scratch.append(pltpu.VMEM_SHARED((NP, WH), jnp.float32))
    if with_cnt:
        scratch.append(pltpu.VMEM_SHARED((NP,), jnp.float32))
    mesh = plsc.VectorSubcoreMesh(core_axis_name="c", subcore_axis_name="s",
                                  num_cores=NC, num_subcores=NS)
    return pl.kernel(functools.partial(_sc_agg_body, P, with_cnt),
                     out_type=out_type, mesh=mesh, scratch_types=scratch)


def _tc_layer1(sums_ref, cnt_ref, x_ref, wl_ref, wr_ref, b_ref, h_ref):
    sm = sums_ref[...]
    agg = sm[0] + sm[1]
    aggr = jnp.concatenate([agg[0], agg[1]], axis=1)
    cnt = cnt_ref[0, 0, :] + cnt_ref[0, 1, :]
    aggr = aggr / jnp.maximum(cnt, 1.0)[:, None]
    out = (jnp.dot(aggr, wl_ref[...], preferred_element_type=jnp.float32)
           + b_ref[...]
           + jnp.dot(x_ref[...], wr_ref[...], preferred_element_type=jnp.float32))
    nrm = jnp.sqrt(jnp.sum(out * out, axis=1, keepdims=True))
    out = out / jnp.maximum(nrm, 1e-12)
    h_ref[...] = jnp.maximum(out, 0.0)


def _tc_layer2(sums_ref, cnt_ref, h_ref, wl_ref, wr_ref, b_ref, batch_ref,
               pooled_ref):
    i = pl.program_id(0)
    sm = sums_ref[...]
    agg = sm[0] + sm[1]
    aggr = jnp.concatenate([agg[0], agg[1], agg[2], agg[3]], axis=1)
    cnt = cnt_ref[0, 0, :] + cnt_ref[0, 1, :]
    aggr = aggr / jnp.maximum(cnt, 1.0)[:, None]
    out = (jnp.dot(aggr, wl_ref[...], preferred_element_type=jnp.float32)
           + b_ref[...]
           + jnp.dot(h_ref[...], wr_ref[...], preferred_element_type=jnp.float32))
    nrm = jnp.sqrt(jnp.sum(out * out, axis=1, keepdims=True))
    out = out / jnp.maximum(nrm, 1e-12)
    out = jnp.maximum(out, 0.0)

    bids = batch_ref[0, 0, :]
    gids = lax.broadcasted_iota(jnp.int32, (G, out.shape[0]), 0)
    mask = (gids == bids[None, :]).astype(jnp.float32)
    part = jnp.dot(mask, out, preferred_element_type=jnp.float32)

    @pl.when(i == 0)
    def _():
        pooled_ref[...] = jnp.zeros_like(pooled_ref)

    pooled_ref[...] += part


def _tc_head(pooled_ref, sfeat_ref, w1a_ref, w1b_ref, b1_ref, w2_ref, b2_ref,
             wp_ref, bp_ref, pred_ref):
    z = (jnp.dot(pooled_ref[...], w1a_ref[...], preferred_element_type=jnp.float32)
         + jnp.dot(sfeat_ref[...], w1b_ref[...], preferred_element_type=jnp.float32)
         + b1_ref[...])
    z = jnp.maximum(z, 0.0)
    z = jnp.dot(z, w2_ref[...], preferred_element_type=jnp.float32) + b2_ref[...]
    z = jnp.maximum(z, 0.0)
    t = jnp.sum(z * wp_ref[...], axis=1, keepdims=True) + bp_ref[...]
    pred_ref[...] = jnp.maximum(-t, 0.0) + jnp.log1p(jnp.exp(-jnp.abs(t)))


def kernel(x, edge_index, batch, static_feature, W1l, b1l, W1r, W2l, b2l, W2r,
           Wfc1, bfc1, Wfc2, bfc2, Wp, bp):
    src = edge_index[0]
    dst = edge_index[1]
    f32 = jnp.float32

    pad = E2 - E
    srcp = jnp.concatenate([src, jnp.zeros((pad,), jnp.int32)])
    dstp = jnp.concatenate([dst, jnp.full((pad,), NP - 1, jnp.int32)])
    dst_idx = dstp.reshape(NW, NB, TB)
    offs2 = jnp.arange(2, dtype=jnp.int32)[:, None] * N
    offs4 = jnp.arange(4, dtype=jnp.int32)[:, None] * N
    src_off1 = (srcp[None, :] + offs2).reshape(2, NW, NB, TB)
    src_off2 = (srcp[None, :] + offs4).reshape(4, NW, NB, TB)
    x_table = x.reshape(N, 2, WH).transpose(1, 0, 2).reshape(2 * N, WH)
    zrows = jnp.zeros((ZB, WH), f32)
    zcnt = jnp.zeros((NP,), f32)
    ones1 = jnp.ones((TB,), f32)

    sums1, cnt = _make_sc_agg(2, True)(x_table, src_off1, dst_idx, zrows,
                                       zcnt, ones1)
    cnt3d = cnt[:, :N].reshape(NC, N // 400, 400).transpose(1, 0, 2)

    R = 400
    grid = (N // R,)
    h = pl.pallas_call(
        _tc_layer1,
        grid=grid,
        in_specs=[
            pl.BlockSpec((NC, 2, R, WH), lambda i: (0, 0, i, 0)),
            pl.BlockSpec((1, NC, R), lambda i: (i, 0, 0)),
            pl.BlockSpec((R, D), lambda i: (i, 0)),
            pl.BlockSpec((D, H), lambda i: (0, 0)),
            pl.BlockSpec((D, H), lambda i: (0, 0)),
            pl.BlockSpec((1, H), lambda i: (0, 0)),
        ],
        out_specs=pl.BlockSpec((R, H), lambda i: (i, 0)),
        out_shape=jax.ShapeDtypeStruct((N, H), f32),
    )(sums1, cnt3d, x, W1l.T, W1r.T, b1l.reshape(1, H))

    h_table = h.reshape(N, 4, WH).transpose(1, 0, 2).reshape(4 * N, WH)
    (sums2,) = _make_sc_agg(4, False)(h_table, src_off2, dst_idx, zrows)

    batch3d = batch.reshape(N // R, 1, R)
    pooled = pl.pallas_call(
        _tc_layer2,
        grid=grid,
        in_specs=[
            pl.BlockSpec((NC, 4, R, WH), lambda i: (0, 0, i, 0)),
            pl.BlockSpec((1, NC, R), lambda i: (i, 0, 0)),
            pl.BlockSpec((R, H), lambda i: (i, 0)),
            pl.BlockSpec((H, H), lambda i: (0, 0)),
            pl.BlockSpec((H, H), lambda i: (0, 0)),
            pl.BlockSpec((1, H), lambda i: (0, 0)),
            pl.BlockSpec((1, 1, R), lambda i: (i, 0, 0)),
        ],
        out_specs=pl.BlockSpec((G, H), lambda i: (0, 0)),
        out_shape=jax.ShapeDtypeStruct((G, H), f32),
    )(sums2, cnt3d, h, W2l.T, W2r.T, b2l.reshape(1, H), batch3d)

    pred = pl.pallas_call(
        _tc_head,
        out_shape=jax.ShapeDtypeStruct((G, 1), f32),
    )(pooled, static_feature, Wfc1[:, :H].T, Wfc1[:, H:].T, bfc1.reshape(1, H),
      Wfc2.T, bfc2.reshape(1, H), Wp, bp.reshape(1, 1))

    return pred


# speedup vs baseline: 1.1164x; 1.1164x over previous
import functools

import jax
import jax.numpy as jnp
from jax import lax
from jax.experimental import pallas as pl
from jax.experimental.pallas import tpu as pltpu
from jax.experimental.pallas import tpu_sc as plsc

N = 10000
E = 160000
D = 256
H = 512
G = 64
SF = 4

NC = 2
NS = 16
NW = NC * NS
WH = 128
TB = 128
NB = 40
E2 = NW * NB * TB
NP = 10240
STRIPE = NP // NS
ZB = 128
ZCH = STRIPE // ZB
CH = 8
NQ = NB // CH


def _sc_agg_body(P, with_cnt, *refs):
    if with_cnt:
        (table, src_off, dst_idx, zrows, ones,
         out_sums, out_cnt,
         srcv, dstv, rows, sems, acc) = refs
    else:
        (table, src_off, dst_idx, zrows,
         out_sums,
         srcv, dstv, rows, sems, acc) = refs

    c = lax.axis_index("c")
    s = lax.axis_index("s")
    tid = s * NC + c
    lo = s * STRIPE

    pltpu.sync_copy(dst_idx.at[tid], dstv)

    passes = (["cnt"] if with_cnt else []) + list(range(P))
    for pp in passes:
        for z in range(ZCH):
            pltpu.sync_copy(zrows, acc.at[pl.ds(lo + z * ZB, ZB)])
        plsc.subcore_barrier()

        sps = [None, None]
        if pp == "cnt":
            pltpu.sync_copy(ones, rows.at[0])
            pltpu.sync_copy(ones, rows.at[1])
            for i in range(NB):
                if sps[i % 2] is not None:
                    sps[i % 2].wait()
                sps[i % 2] = pltpu.async_copy(rows.at[i % 2],
                                              acc.at[dstv.at[i]],
                                              sems[2 + i % 2], add=True)
        else:
            pltpu.sync_copy(src_off.at[pp, tid], srcv)
            gps = [None, None]
            gps[0] = pltpu.async_copy(table.at[srcv.at[0]], rows.at[0],
                                      sems[0])
            for i in range(NB):
                if i + 1 < NB:
                    if sps[(i + 1) % 2] is not None:
                        sps[(i + 1) % 2].wait()
                        sps[(i + 1) % 2] = None
                    gps[(i + 1) % 2] = pltpu.async_copy(
                        table.at[srcv.at[i + 1]], rows.at[(i + 1) % 2],
                        sems[(i + 1) % 2])
                gps[i % 2].wait()
                sps[i % 2] = pltpu.async_copy(rows.at[i % 2],
                                              acc.at[dstv.at[i]],
                                              sems[2 + i % 2], add=True)
        for j in range(2):
            if sps[j] is not None:
                sps[j].wait()
        plsc.subcore_barrier()
        dst_out = out_cnt.at[c] if pp == "cnt" else out_sums.at[c, pp]
        for z in range(ZCH):
            pltpu.sync_copy(acc.at[pl.ds(lo + z * ZB, ZB)],
                            dst_out.at[pl.ds(lo + z * ZB, ZB)])


def _make_sc_agg(P, with_cnt):
    out_type = [jax.ShapeDtypeStruct((NC, P, NP, WH), jnp.float32)]
    if with_cnt:
        out_type.append(jax.ShapeDtypeStruct((NC, NP, WH), jnp.float32))
    scratch = [
        pltpu.VMEM((NB, TB), jnp.int32),
        pltpu.VMEM((NB, TB), jnp.int32),
        pltpu.VMEM((2, TB, WH), jnp.float32),
        [pltpu.SemaphoreType.DMA] * 4,
        pltpu.VMEM_SHARED((NP, WH), jnp.float32),
    ]
    mesh = plsc.VectorSubcoreMesh(core_axis_name="c", subcore_axis_name="s",
                                  num_cores=NC, num_subcores=NS)
    return pl.kernel(functools.partial(_sc_agg_body, P, with_cnt),
                     out_type=out_type, mesh=mesh, scratch_types=scratch)


def _tc_layer1(sums_ref, cnt_ref, x_ref, wl_ref, wr_ref, b_ref, h_ref):
    sm = sums_ref[...]
    agg = sm[0] + sm[1]
    aggr = jnp.concatenate([agg[0], agg[1]], axis=1)
    cnt = cnt_ref[0, :, 0] + cnt_ref[1, :, 0]
    aggr = aggr / jnp.maximum(cnt, 1.0)[:, None]
    out = (jnp.dot(aggr, wl_ref[...], preferred_element_type=jnp.float32)
           + b_ref[...]
           + jnp.dot(x_ref[...], wr_ref[...], preferred_element_type=jnp.float32))
    nrm = jnp.sqrt(jnp.sum(out * out, axis=1, keepdims=True))
    out = out / jnp.maximum(nrm, 1e-12)
    h_ref[...] = jnp.maximum(out, 0.0)


def _tc_layer2(sums_ref, cnt_ref, h_ref, wl_ref, wr_ref, b_ref, batch_ref,
               pooled_ref):
    i = pl.program_id(0)
    sm = sums_ref[...]
    agg = sm[0] + sm[1]
    aggr = jnp.concatenate([agg[0], agg[1], agg[2], agg[3]], axis=1)
    cnt = cnt_ref[0, :, 0] + cnt_ref[1, :, 0]
    aggr = aggr / jnp.maximum(cnt, 1.0)[:, None]
    out = (jnp.dot(aggr, wl_ref[...], preferred_element_type=jnp.float32)
           + b_ref[...]
           + jnp.dot(h_ref[...], wr_ref[...], preferred_element_type=jnp.float32))
    nrm = jnp.sqrt(jnp.sum(out * out, axis=1, keepdims=True))
    out = out / jnp.maximum(nrm, 1e-12)
    out = jnp.maximum(out, 0.0)

    bids = batch_ref[0, 0, :]
    gids = lax.broadcasted_iota(jnp.int32, (G, out.shape[0]), 0)
    mask = (gids == bids[None, :]).astype(jnp.float32)
    part = jnp.dot(mask, out, preferred_element_type=jnp.float32)

    @pl.when(i == 0)
    def _():
        pooled_ref[...] = jnp.zeros_like(pooled_ref)

    pooled_ref[...] += part


def _tc_head(pooled_ref, sfeat_ref, w1a_ref, w1b_ref, b1_ref, w2_ref, b2_ref,
             wp_ref, bp_ref, pred_ref):
    z = (jnp.dot(pooled_ref[...], w1a_ref[...], preferred_element_type=jnp.float32)
         + jnp.dot(sfeat_ref[...], w1b_ref[...], preferred_element_type=jnp.float32)
         + b1_ref[...])
    z = jnp.maximum(z, 0.0)
    z = jnp.dot(z, w2_ref[...], preferred_element_type=jnp.float32) + b2_ref[...]
    z = jnp.maximum(z, 0.0)
    t = jnp.sum(z * wp_ref[...], axis=1, keepdims=True) + bp_ref[...]
    pred_ref[...] = jnp.maximum(-t, 0.0) + jnp.log1p(jnp.exp(-jnp.abs(t)))


def kernel(x, edge_index, batch, static_feature, W1l, b1l, W1r, W2l, b2l, W2r,
           Wfc1, bfc1, Wfc2, bfc2, Wp, bp):
    src = edge_index[0]
    dst = edge_index[1]
    f32 = jnp.float32

    pad = E2 - E
    srcp = jnp.concatenate([src, jnp.zeros((pad,), jnp.int32)])
    dstp = jnp.concatenate([dst, jnp.full((pad,), NP - 1, jnp.int32)])
    dst_idx = dstp.reshape(NW, NB, TB)
    offs2 = jnp.arange(2, dtype=jnp.int32)[:, None] * N
    offs4 = jnp.arange(4, dtype=jnp.int32)[:, None] * N
    src_off1 = (srcp[None, :] + offs2).reshape(2, NW, NB, TB)
    src_off2 = (srcp[None, :] + offs4).reshape(4, NW, NB, TB)
    x_table = x.reshape(N, 2, WH).transpose(1, 0, 2).reshape(2 * N, WH)
    zrows = jnp.zeros((ZB, WH), f32)
    ones = jnp.ones((TB, WH), f32)

    sums1, cnt = _make_sc_agg(2, True)(x_table, src_off1, dst_idx, zrows,
                                       ones)

    R = 400
    grid = (N // R,)
    h = pl.pallas_call(
        _tc_layer1,
        grid=grid,
        in_specs=[
            pl.BlockSpec((NC, 2, R, WH), lambda i: (0, 0, i, 0)),
            pl.BlockSpec((NC, R, WH), lambda i: (0, i, 0)),
            pl.BlockSpec((R, D), lambda i: (i, 0)),
            pl.BlockSpec((D, H), lambda i: (0, 0)),
            pl.BlockSpec((D, H), lambda i: (0, 0)),
            pl.BlockSpec((1, H), lambda i: (0, 0)),
        ],
        out_specs=pl.BlockSpec((R, H), lambda i: (i, 0)),
        out_shape=jax.ShapeDtypeStruct((N, H), f32),
    )(sums1, cnt, x, W1l.T, W1r.T, b1l.reshape(1, H))

    h_table = h.reshape(N, 4, WH).transpose(1, 0, 2).reshape(4 * N, WH)
    (sums2,) = _make_sc_agg(4, False)(h_table, src_off2, dst_idx, zrows)

    batch3d = batch.reshape(N // R, 1, R)
    pooled = pl.pallas_call(
        _tc_layer2,
        grid=grid,
        in_specs=[
            pl.BlockSpec((NC, 4, R, WH), lambda i: (0, 0, i, 0)),
            pl.BlockSpec((NC, R, WH), lambda i: (0, i, 0)),
            pl.BlockSpec((R, H), lambda i: (i, 0)),
            pl.BlockSpec((H, H), lambda i: (0, 0)),
            pl.BlockSpec((H, H), lambda i: (0, 0)),
            pl.BlockSpec((1, H), lambda i: (0, 0)),
            pl.BlockSpec((1, 1, R), lambda i: (i, 0, 0)),
        ],
        out_specs=pl.BlockSpec((G, H), lambda i: (0, 0)),
        out_shape=jax.ShapeDtypeStruct((G, H), f32),
    )(sums2, cnt, h, W2l.T, W2r.T, b2l.reshape(1, H), batch3d)

    pred = pl.pallas_call(
        _tc_head,
        out_shape=jax.ShapeDtypeStruct((G, 1), f32),
    )(pooled, static_feature, Wfc1[:, :H].T, Wfc1[:, H:].T, bfc1.reshape(1, H),
      Wfc2.T, bfc2.reshape(1, H), Wp, bp.reshape(1, 1))

    return pred
